# Initial kernel scaffold; baseline (speedup 1.0000x reference)
import jax
import jax.numpy as jnp
from jax.experimental import pallas as pl


def kernel(x, embeddings):
    raise NotImplementedError("write your pallas kernel here")


# speedup vs baseline: 1.7069x; 1.7069x over previous
import functools

import jax
import jax.numpy as jnp
from jax import lax
from jax.experimental import pallas as pl
from jax.experimental.pallas import tpu as pltpu
from jax.experimental.pallas import tpu_sc as plsc

V_DIM = 1000000
EMB = 32
BATCH = 16384
CTX = 20

NC = 2
NS = 16
NW = NC * NS
BPW = BATCH // NW
CB = 64
STEPS = BPW // CB
IPC = CB * CTX
GW = 128
GROWS = IPC // GW
IDX_ROWS = BATCH * CTX // GW


def _cbow_body(x_hbm, tab_hbm, out_hbm, idx_v, rows_v, acc_v, sem):
    wid = lax.axis_index("s") * NC + lax.axis_index("c")
    pltpu.sync_copy(x_hbm.at[pl.ds(wid * (STEPS * GROWS), STEPS * GROWS)], idx_v)

    def step(s, carry):
        cps = [
            pltpu.async_copy(
                tab_hbm.at[idx_v.at[s * GROWS + j]],
                rows_v.at[pl.ds(j * GW, GW)],
                sem,
            )
            for j in range(GROWS)
        ]
        for cp in cps:
            cp.wait()

        def comp(b, c2):
            base = b * CTX
            acc0 = jnp.zeros((16,), jnp.float32)
            acc1 = jnp.zeros((16,), jnp.float32)
            for c in range(CTX):
                acc0 = acc0 + rows_v[base + c, 0:16]
                acc1 = acc1 + rows_v[base + c, 16:32]
            acc_v[b, 0:16] = acc0 * (1.0 / CTX)
            acc_v[b, 16:32] = acc1 * (1.0 / CTX)
            return c2

        lax.fori_loop(0, CB, comp, 0)
        pltpu.sync_copy(acc_v, out_hbm.at[pl.ds(wid * BPW + s * CB, CB)])
        return carry

    lax.fori_loop(0, STEPS, step, 0)


_cbow = functools.partial(
    pl.kernel,
    out_type=jax.ShapeDtypeStruct((BATCH, EMB), jnp.float32),
    mesh=plsc.VectorSubcoreMesh(core_axis_name="c", subcore_axis_name="s"),
    compiler_params=pltpu.CompilerParams(use_tc_tiling_on_sc=False),
    scratch_types=[
        pltpu.VMEM((STEPS * GROWS, GW), jnp.int32),
        pltpu.VMEM((IPC, EMB), jnp.float32),
        pltpu.VMEM((CB, EMB), jnp.float32),
        pltpu.SemaphoreType.DMA,
    ],
)(_cbow_body)


def kernel(x, embeddings):
    x2 = x.reshape(IDX_ROWS, GW).astype(jnp.int32)
    return _cbow(x2, embeddings)


# speedup vs baseline: 1.7162x; 1.0055x over previous
import functools

import jax
import jax.numpy as jnp
from jax import lax
from jax.experimental import pallas as pl
from jax.experimental.pallas import tpu as pltpu
from jax.experimental.pallas import tpu_sc as plsc

V_DIM = 1000000
EMB = 32
BATCH = 16384
CTX = 20

NC = 2
NS = 16
NW = NC * NS
BPW = BATCH // NW
CB = 128
STEPS = BPW // CB
IPC = CB * CTX
GW = 128
GROWS = IPC // GW
IDX_ROWS = BATCH * CTX // GW


def _cbow_body(x_hbm, tab_hbm, out_hbm, idx_v, rows_v, acc_v, sem):
    wid = lax.axis_index("s") * NC + lax.axis_index("c")
    pltpu.sync_copy(x_hbm.at[pl.ds(wid * (STEPS * GROWS), STEPS * GROWS)], idx_v)

    def step(s, carry):
        cps = [
            pltpu.async_copy(
                tab_hbm.at[idx_v.at[s * GROWS + j]],
                rows_v.at[pl.ds(j * GW, GW)],
                sem,
            )
            for j in range(GROWS)
        ]
        for cp in cps:
            cp.wait()

        def comp(b, c2):
            base = b * CTX
            acc0 = jnp.zeros((16,), jnp.float32)
            acc1 = jnp.zeros((16,), jnp.float32)
            for c in range(CTX):
                acc0 = acc0 + rows_v[base + c, 0:16]
                acc1 = acc1 + rows_v[base + c, 16:32]
            acc_v[b, 0:16] = acc0 * (1.0 / CTX)
            acc_v[b, 16:32] = acc1 * (1.0 / CTX)
            return c2

        lax.fori_loop(0, CB, comp, 0)
        pltpu.sync_copy(acc_v, out_hbm.at[pl.ds(wid * BPW + s * CB, CB)])
        return carry

    lax.fori_loop(0, STEPS, step, 0)


_cbow = functools.partial(
    pl.kernel,
    out_type=jax.ShapeDtypeStruct((BATCH, EMB), jnp.float32),
    mesh=plsc.VectorSubcoreMesh(core_axis_name="c", subcore_axis_name="s"),
    compiler_params=pltpu.CompilerParams(use_tc_tiling_on_sc=False),
    scratch_types=[
        pltpu.VMEM((STEPS * GROWS, GW), jnp.int32),
        pltpu.VMEM((IPC, EMB), jnp.float32),
        pltpu.VMEM((CB, EMB), jnp.float32),
        pltpu.SemaphoreType.DMA,
    ],
)(_cbow_body)


def kernel(x, embeddings):
    x2 = x.reshape(IDX_ROWS, GW).astype(jnp.int32)
    return _cbow(x2, embeddings)
